# baseline (speedup 1.0000x reference)
import functools

import jax
import jax.numpy as jnp
from jax import lax
from jax.experimental import pallas as pl
from jax.experimental.pallas import tpu as pltpu
from jax.experimental.pallas import tpu_sc as plsc

H = 512
W = 512
NPH = 32
NPW = 32
PH = H // NPH
PW = W // NPW
NP = NPH * NPW
C = 192

NC = 2
NS = 16
HP = PH // 2
NBUF = 4

NPH_SC = 20
NCH = NPH_SC * 2


def _sc_body(x_hbm, out_hbm, bufs, sls, sss):
    wid = lax.axis_index("s") * NC + lax.axis_index("c")
    tbase = wid * NCH

    def coords(ch):
        gch = tbase + ch
        nph = lax.div(gch, 2 * NPW)
        rem = lax.rem(gch, 2 * NPW)
        j = lax.div(rem, 2)
        jh = lax.rem(rem, 2)
        return nph, j, jh

    def src(ch):
        nph, j, jh = coords(ch)
        return x_hbm.at[
            pl.ds(nph * PH + jh * HP, HP), pl.ds(j * PW, PW), :
        ]

    def dst(ch):
        nph, j, jh = coords(ch)
        return out_hbm.at[nph * NPW + j, pl.ds(jh * HP, HP)]

    def load(ch, b):
        pltpu.async_copy(src(ch), bufs[b], sls[b])

    def store(ch, b):
        pltpu.async_copy(bufs[b], dst(ch), sss[b])

    def wait_l(b):
        pltpu.make_async_copy(src(0), bufs[b], sls[b]).wait()

    def wait_s(b):
        pltpu.make_async_copy(bufs[b], dst(0), sss[b]).wait()

    for b in range(NBUF):
        load(b, b)

    def step(i, _):
        ch = NBUF * i
        for b in range(NBUF):
            wait_l(b)
            store(ch + b, b)
        for b in range(NBUF):
            wait_s(b)
            load(ch + NBUF + b, b)
        return 0

    lax.fori_loop(0, NCH // NBUF - 1, step, 0)

    ch = NCH - NBUF
    for b in range(NBUF):
        wait_l(b)
        store(ch + b, b)
    for b in range(NBUF):
        wait_s(b)


_sc_kernel = functools.partial(
    pl.kernel,
    out_type=jax.ShapeDtypeStruct((NP, PH, PW, C), jnp.float32),
    mesh=plsc.VectorSubcoreMesh(
        core_axis_name="c", subcore_axis_name="s", num_cores=NC, num_subcores=NS
    ),
    scratch_types=[
        [pltpu.VMEM((HP, PW, C), jnp.float32) for _ in range(NBUF)],
        [pltpu.SemaphoreType.DMA for _ in range(NBUF)],
        [pltpu.SemaphoreType.DMA for _ in range(NBUF)],
    ],
    compiler_params=pltpu.CompilerParams(use_tc_tiling_on_sc=True),
)(_sc_body)


def _tc_body(x_ref, y_ref, o_ref, sem):
    nph = NPH_SC + pl.program_id(0)
    for j in range(NPW):
        pltpu.make_async_copy(
            x_ref.at[:, pl.ds(j * PW, PW), :],
            o_ref.at[nph * NPW + j],
            sem,
        ).start()
    for j in range(NPW):
        pltpu.make_async_copy(
            x_ref.at[:, pl.ds(0, PW), :],
            o_ref.at[0],
            sem,
        ).wait()


def _tc_kernel(x3, y):
    return pl.pallas_call(
        _tc_body,
        grid=(NPH - NPH_SC,),
        in_specs=[
            pl.BlockSpec((PH, W, C), lambda i: (NPH_SC + i, 0, 0)),
            pl.BlockSpec(memory_space=pl.ANY),
        ],
        out_specs=pl.BlockSpec(memory_space=pl.ANY),
        out_shape=jax.ShapeDtypeStruct((NP, PH, PW, C), jnp.float32),
        input_output_aliases={1: 0},
        scratch_shapes=[pltpu.SemaphoreType.DMA],
    )(x3, y)


def kernel(x, mesh_pos, batch_idx):
    x3 = x.reshape(H, W, C)
    out = _sc_kernel(x3)
    out = _tc_kernel(x3, out)
    return out.reshape(1, NP, PH, PW, C)


# speedup vs baseline: 1.0033x; 1.0033x over previous
import functools

import jax
import jax.numpy as jnp
from jax import lax
from jax.experimental import pallas as pl
from jax.experimental.pallas import tpu as pltpu
from jax.experimental.pallas import tpu_sc as plsc

H = 512
W = 512
NPH = 32
NPW = 32
PH = H // NPH
PW = W // NPW
NP = NPH * NPW
C = 192

NC = 2
NS = 16
HP = PH // 2
NBUF = 4

NPH_SC = 12
NCH = NPH_SC * 2


def _sc_body(x_hbm, out_hbm, bufs, sls, sss):
    wid = lax.axis_index("s") * NC + lax.axis_index("c")
    tbase = wid * NCH

    def coords(ch):
        gch = tbase + ch
        nph = lax.div(gch, 2 * NPW)
        rem = lax.rem(gch, 2 * NPW)
        j = lax.div(rem, 2)
        jh = lax.rem(rem, 2)
        return nph, j, jh

    def src(ch):
        nph, j, jh = coords(ch)
        return x_hbm.at[
            pl.ds(nph * PH + jh * HP, HP), pl.ds(j * PW, PW), :
        ]

    def dst(ch):
        nph, j, jh = coords(ch)
        return out_hbm.at[nph * NPW + j, pl.ds(jh * HP, HP)]

    def load(ch, b):
        pltpu.async_copy(src(ch), bufs[b], sls[b])

    def store(ch, b):
        pltpu.async_copy(bufs[b], dst(ch), sss[b])

    def wait_l(b):
        pltpu.make_async_copy(src(0), bufs[b], sls[b]).wait()

    def wait_s(b):
        pltpu.make_async_copy(bufs[b], dst(0), sss[b]).wait()

    for b in range(NBUF):
        load(b, b)

    def step(i, _):
        ch = NBUF * i
        for b in range(NBUF):
            wait_l(b)
            store(ch + b, b)
        for b in range(NBUF):
            wait_s(b)
            load(ch + NBUF + b, b)
        return 0

    lax.fori_loop(0, NCH // NBUF - 1, step, 0)

    ch = NCH - NBUF
    for b in range(NBUF):
        wait_l(b)
        store(ch + b, b)
    for b in range(NBUF):
        wait_s(b)


_sc_kernel = functools.partial(
    pl.kernel,
    out_type=jax.ShapeDtypeStruct((NP, PH, PW, C), jnp.float32),
    mesh=plsc.VectorSubcoreMesh(
        core_axis_name="c", subcore_axis_name="s", num_cores=NC, num_subcores=NS
    ),
    scratch_types=[
        [pltpu.VMEM((HP, PW, C), jnp.float32) for _ in range(NBUF)],
        [pltpu.SemaphoreType.DMA for _ in range(NBUF)],
        [pltpu.SemaphoreType.DMA for _ in range(NBUF)],
    ],
    compiler_params=pltpu.CompilerParams(use_tc_tiling_on_sc=True),
)(_sc_body)


def _tc_body(x_ref, y_ref, o_ref, sem):
    nph = NPH_SC + pl.program_id(0)
    for j in range(NPW):
        pltpu.make_async_copy(
            x_ref.at[:, pl.ds(j * PW, PW), :],
            o_ref.at[nph * NPW + j],
            sem,
        ).start()
    for j in range(NPW):
        pltpu.make_async_copy(
            x_ref.at[:, pl.ds(0, PW), :],
            o_ref.at[0],
            sem,
        ).wait()


def _tc_kernel(x3, y):
    return pl.pallas_call(
        _tc_body,
        grid=(NPH - NPH_SC,),
        in_specs=[
            pl.BlockSpec((PH, W, C), lambda i: (NPH_SC + i, 0, 0)),
            pl.BlockSpec(memory_space=pl.ANY),
        ],
        out_specs=pl.BlockSpec(memory_space=pl.ANY),
        out_shape=jax.ShapeDtypeStruct((NP, PH, PW, C), jnp.float32),
        input_output_aliases={1: 0},
        scratch_shapes=[pltpu.SemaphoreType.DMA],
    )(x3, y)


def kernel(x, mesh_pos, batch_idx):
    x3 = x.reshape(H, W, C)
    out = _sc_kernel(x3)
    out = _tc_kernel(x3, out)
    return out.reshape(1, NP, PH, PW, C)
